# Initial kernel scaffold; baseline (speedup 1.0000x reference)
import functools

import jax
import jax.numpy as jnp
from jax import lax
from jax.experimental import pallas as pl
from jax.experimental.pallas import tpu as pltpu
from jax.experimental.pallas import tpu_sc as plsc

NC, NS = 2, 16
NW = NC * NS
K = 80


def _sc_layer(N, DN, DE, E):
    assert E % (NW * K) == 0 and N % NS == 0
    nch = E // (NW * K)
    epw = nch * K
    rpt = N // NS
    mesh = plsc.VectorSubcoreMesh(core_axis_name="c", subcore_axis_name="s",
                                  num_cores=NC, num_subcores=NS)
    f32 = jnp.float32

    @functools.partial(
        pl.kernel, mesh=mesh,
        out_type=[
            jax.ShapeDtypeStruct((NC, N, DN), f32),
            jax.ShapeDtypeStruct((NC, N, DE), f32),
            jax.ShapeDtypeStruct((NC, N, DE), f32),
            jax.ShapeDtypeStruct((E, DE), f32),
            jax.ShapeDtypeStruct((E, DE), f32),
        ],
        scratch_types=[
            pltpu.VMEM_SHARED((N + 8, DN), f32),
            pltpu.VMEM_SHARED((N + 8, DE), f32),
            pltpu.VMEM_SHARED((N + 8, DE), f32),
            pltpu.VMEM((nch, K), jnp.int32),
            pltpu.VMEM((nch, K), jnp.int32),
            pltpu.VMEM((K, DN), f32),
            pltpu.VMEM((K, DE), f32),
            pltpu.VMEM((K, DE), f32),
            pltpu.VMEM((K, DE), f32),
            pltpu.VMEM((K, DE), f32),
            pltpu.SemaphoreType.DMA,
            pltpu.SemaphoreType.DMA,
            pltpu.SemaphoreType.DMA,
        ],
    )
    def sc_kernel(x_hbm, pu_hbm, pv_hbm, e_hbm, src_hbm, dst_hbm, zx_hbm,
                  ze_hbm, ones_hbm,
                  accx_hbm, acce_hbm, accd_hbm, gu_hbm, gv_hbm,
                  shx, she, shd, src_v, dst_v, rows_v, ev_v, ones_v,
                  gu_v, gv_v, sem0, sem1, sem2):
        c = lax.axis_index("c")
        s = lax.axis_index("s")
        wid = s * NC + c
        r0 = s * rpt
        pltpu.sync_copy(zx_hbm.at[pl.ds(r0, rpt)], shx.at[pl.ds(r0, rpt)])
        pltpu.sync_copy(ze_hbm.at[pl.ds(r0, rpt)], she.at[pl.ds(r0, rpt)])
        pltpu.sync_copy(ze_hbm.at[pl.ds(r0, rpt)], shd.at[pl.ds(r0, rpt)])
        pltpu.sync_copy(ones_hbm, ones_v)
        pltpu.sync_copy(src_hbm.at[wid], src_v)
        pltpu.sync_copy(dst_hbm.at[wid], dst_v)
        plsc.subcore_barrier()

        ebase = wid * epw

        def body(j, carry):
            src_row = src_v.at[j]
            dst_row = dst_v.at[j]
            eo = ebase + j * K
            ga = pltpu.async_copy(x_hbm.at[src_row], rows_v, sem0)
            gb = pltpu.async_copy(pu_hbm.at[src_row], gu_v, sem1)
            gc = pltpu.async_copy(pv_hbm.at[dst_row], gv_v, sem2)
            pltpu.sync_copy(e_hbm.at[pl.ds(eo, K)], ev_v)
            ga.wait()
            pltpu.sync_copy(rows_v, shx.at[dst_row], add=True)
            pltpu.sync_copy(ev_v, she.at[dst_row], add=True)
            pltpu.sync_copy(ones_v, shd.at[dst_row], add=True)
            gb.wait()
            gc.wait()
            pltpu.sync_copy(gu_v, gu_hbm.at[pl.ds(eo, K)])
            pltpu.sync_copy(gv_v, gv_hbm.at[pl.ds(eo, K)])
            return carry

        lax.fori_loop(0, nch, body, 0)
        plsc.subcore_barrier()
        pltpu.sync_copy(shx.at[pl.ds(r0, rpt)], accx_hbm.at[c, pl.ds(r0, rpt)])
        pltpu.sync_copy(she.at[pl.ds(r0, rpt)], acce_hbm.at[c, pl.ds(r0, rpt)])
        pltpu.sync_copy(shd.at[pl.ds(r0, rpt)], accd_hbm.at[c, pl.ds(r0, rpt)])

    return sc_kernel


def _sc_head(N, PH, E):
    assert E % (NW * K) == 0
    nch = E // (NW * K)
    epw = nch * K
    mesh = plsc.VectorSubcoreMesh(core_axis_name="c", subcore_axis_name="s",
                                  num_cores=NC, num_subcores=NS)
    f32 = jnp.float32

    @functools.partial(
        pl.kernel, mesh=mesh,
        out_type=[
            jax.ShapeDtypeStruct((E, PH), f32),
            jax.ShapeDtypeStruct((E, PH), f32),
        ],
        scratch_types=[
            pltpu.VMEM((nch, K), jnp.int32),
            pltpu.VMEM((nch, K), jnp.int32),
            pltpu.VMEM((K, PH), f32),
            pltpu.VMEM((K, PH), f32),
            pltpu.SemaphoreType.DMA,
            pltpu.SemaphoreType.DMA,
        ],
    )
    def sc_kernel(pa_hbm, pb_hbm, src_hbm, dst_hbm, ga_hbm, gb_hbm,
                  src_v, dst_v, ga_v, gb_v, sem0, sem1):
        c = lax.axis_index("c")
        s = lax.axis_index("s")
        wid = s * NC + c
        pltpu.sync_copy(src_hbm.at[wid], src_v)
        pltpu.sync_copy(dst_hbm.at[wid], dst_v)
        ebase = wid * epw

        def body(j, carry):
            eo = ebase + j * K
            ga = pltpu.async_copy(pa_hbm.at[src_v.at[j]], ga_v, sem0)
            gb = pltpu.async_copy(pb_hbm.at[dst_v.at[j]], gb_v, sem1)
            ga.wait()
            pltpu.sync_copy(ga_v, ga_hbm.at[pl.ds(eo, K)])
            gb.wait()
            pltpu.sync_copy(gb_v, gb_hbm.at[pl.ds(eo, K)])
            return carry

        lax.fori_loop(0, nch, body, 0)

    return sc_kernel


def _full(shape):
    return pl.BlockSpec(shape, lambda i: (0,) * len(shape))


def _tc_proj(N, DN, PH, BN=1000):
    nb = N // BN

    def body(x_ref, w_ref, o1_ref, o2_ref):
        p = jnp.dot(x_ref[...], w_ref[...], preferred_element_type=jnp.float32)
        o1_ref[...] = p[:, :PH]
        o2_ref[...] = p[:, PH:]

    return pl.pallas_call(
        body, grid=(nb,),
        in_specs=[pl.BlockSpec((BN, DN), lambda i: (i, 0)),
                  _full((DN, 2 * PH))],
        out_specs=[pl.BlockSpec((BN, PH), lambda i: (i, 0)),
                   pl.BlockSpec((BN, PH), lambda i: (i, 0))],
        out_shape=[jax.ShapeDtypeStruct((N, PH), jnp.float32),
                   jax.ShapeDtypeStruct((N, PH), jnp.float32)],
    )


def _tc_node(N, DN, DE, PH, BN=1000):
    nb = N // BN

    def body(ax_ref, ae_ref, ad_ref, x_ref, wn_ref, we_ref, ws_ref,
             bnbe_ref, b_ref, wpost_ref, nx_ref, p1_ref, p2_ref):
        ax = ax_ref[0] + ax_ref[1]
        ae = ae_ref[0] + ae_ref[1]
        deg = (ad_ref[0] + ad_ref[1])[:, 0:1]
        h = (jnp.dot(ax, wn_ref[...], preferred_element_type=jnp.float32)
             + jnp.dot(ae, we_ref[...], preferred_element_type=jnp.float32)
             + deg * bnbe_ref[...]
             + jnp.dot(x_ref[...], ws_ref[...],
                       preferred_element_type=jnp.float32)
             + b_ref[...])
        nx = jnp.maximum(h, 0.0)
        nx_ref[...] = nx
        p = jnp.dot(nx, wpost_ref[...], preferred_element_type=jnp.float32)
        p1_ref[...] = p[:, :PH]
        p2_ref[...] = p[:, PH:]

    return pl.pallas_call(
        body, grid=(nb,),
        in_specs=[pl.BlockSpec((NC, BN, DN), lambda i: (0, i, 0)),
                  pl.BlockSpec((NC, BN, DE), lambda i: (0, i, 0)),
                  pl.BlockSpec((NC, BN, DE), lambda i: (0, i, 0)),
                  pl.BlockSpec((BN, DN), lambda i: (i, 0)),
                  _full((DN, DN)), _full((DE, DN)), _full((DN, DN)),
                  _full((1, DN)), _full((1, DN)), _full((DN, 2 * PH))],
        out_specs=[pl.BlockSpec((BN, DN), lambda i: (i, 0)),
                   pl.BlockSpec((BN, PH), lambda i: (i, 0)),
                   pl.BlockSpec((BN, PH), lambda i: (i, 0))],
        out_shape=[jax.ShapeDtypeStruct((N, DN), jnp.float32),
                   jax.ShapeDtypeStruct((N, PH), jnp.float32),
                   jax.ShapeDtypeStruct((N, PH), jnp.float32)],
    )


def _tc_edge(E, DE, BE=8000):
    nb = E // BE

    def body(gu_ref, gv_ref, e_ref, w_ref, b_ref, out_ref):
        h = (gu_ref[...] + gv_ref[...]
             + jnp.dot(e_ref[...], w_ref[...],
                       preferred_element_type=jnp.float32)
             + b_ref[...])
        out_ref[...] = jnp.maximum(h, 0.0)

    return pl.pallas_call(
        body, grid=(nb,),
        in_specs=[pl.BlockSpec((BE, DE), lambda i: (i, 0)),
                  pl.BlockSpec((BE, DE), lambda i: (i, 0)),
                  pl.BlockSpec((BE, DE), lambda i: (i, 0)),
                  _full((DE, DE)), _full((1, DE))],
        out_specs=pl.BlockSpec((BE, DE), lambda i: (i, 0)),
        out_shape=jax.ShapeDtypeStruct((E, DE), jnp.float32),
    )


def _tc_head(E, DE, PH, HID, BE=4000):
    nb = E // BE

    def body(ga_ref, gb_ref, e_ref, w1e_ref, b1_ref, w2_ref, b2_ref,
             w3_ref, b3_ref, out_ref):
        h1 = (ga_ref[...] + gb_ref[...]
              + jnp.dot(e_ref[...], w1e_ref[...],
                        preferred_element_type=jnp.float32)
              + b1_ref[...])
        h1 = jnp.maximum(h1, 0.0)
        h2 = jnp.dot(h1, w2_ref[...], preferred_element_type=jnp.float32)
        h2 = jnp.maximum(h2 + b2_ref[...], 0.0)
        z = jnp.dot(h2, w3_ref[...], preferred_element_type=jnp.float32)
        out_ref[...] = jax.nn.sigmoid(z + b3_ref[...])

    return pl.pallas_call(
        body, grid=(nb,),
        in_specs=[pl.BlockSpec((BE, PH), lambda i: (i, 0)),
                  pl.BlockSpec((BE, PH), lambda i: (i, 0)),
                  pl.BlockSpec((BE, DE), lambda i: (i, 0)),
                  _full((DE, PH)), _full((1, PH)), _full((PH, HID)),
                  _full((1, HID)), _full((HID, 1)), _full((1, 1))],
        out_specs=pl.BlockSpec((BE, 1), lambda i: (i, 0)),
        out_shape=jax.ShapeDtypeStruct((E, 1), jnp.float32),
    )


def kernel(x, edge_index, edge_attr, params):
    N, DN = x.shape
    E = edge_index.shape[1]
    DE = edge_attr.shape[1]
    HID = params["mlp"]["W2"].shape[0]
    f32 = jnp.float32

    src = edge_index[0].astype(jnp.int32)
    dst = edge_index[1].astype(jnp.int32)
    nch = E // (NW * K)
    src3 = src.reshape(NW, nch, K)
    dst3 = dst.reshape(NW, nch, K)

    zx = jnp.zeros((N, DN), f32)
    ze = jnp.zeros((N, DE), f32)
    ones_k = jnp.ones((K, DE), f32)

    sc_layer = _sc_layer(N, DN, DE, E)
    tc_node = _tc_node(N, DN, DE, DE)
    tc_edge = _tc_edge(E, DE)

    layers = params["layers"]
    wu0 = layers[0]["Wu"]
    pu, pv = _tc_proj(N, DN, DE)(x, wu0[: 2 * DN])

    xcur, ecur = x, edge_attr
    for li, p in enumerate(layers):
        accx, acce, accd, gu, gv = sc_layer(
            xcur, pu, pv, ecur, src3, dst3, zx, ze, ones_k)
        bnbe = (p["bn"] + p["be"]).reshape(1, DN)
        bnode = (p["bs"] + p["node_scalar"]).reshape(1, DN)
        if li + 1 < len(layers):
            wpost = layers[li + 1]["Wu"][: 2 * DN]
            newx, pu2, pv2 = tc_node(
                accx, acce, accd, xcur, p["Wn"], p["We"], p["Ws"],
                bnbe, bnode, wpost)
        else:
            wpost = params["mlp"]["W1"][: 2 * DN]
            newx, pu2, pv2 = _tc_node(N, DN, DE, HID)(
                accx, acce, accd, xcur, p["Wn"], p["We"], p["Ws"],
                bnbe, bnode, wpost)
        bedge = (p["bu"] + p["edge_scalar"]).reshape(1, DE)
        newe = tc_edge(gu, gv, ecur, p["Wu"][2 * DN:], bedge)
        xcur, ecur, pu, pv = newx, newe, pu2, pv2

    m = params["mlp"]
    ga, gb = _sc_head(N, HID, E)(pu, pv, src3, dst3)
    out = _tc_head(E, DE, HID, HID)(
        ga, gb, ecur, m["W1"][2 * DN:], m["b1"].reshape(1, HID),
        m["W2"], m["b2"].reshape(1, HID), m["W3"], m["b3"].reshape(1, 1))
    return out.reshape(E)


# speedup vs baseline: 1.3354x; 1.3354x over previous
import functools

import jax
import jax.numpy as jnp
from jax import lax
from jax.experimental import pallas as pl
from jax.experimental.pallas import tpu as pltpu
from jax.experimental.pallas import tpu_sc as plsc

NC, NS = 2, 16
NW = NC * NS
K = 80
BF = jnp.bfloat16
F32 = jnp.float32


def _mesh():
    return plsc.VectorSubcoreMesh(core_axis_name="c", subcore_axis_name="s",
                                  num_cores=NC, num_subcores=NS)


def _sc_gather2(N, DN, E):
    assert E % (NW * K) == 0
    nch = E // (NW * K)
    epw = nch * K

    @functools.partial(
        pl.kernel, mesh=_mesh(),
        compiler_params=pltpu.CompilerParams(use_tc_tiling_on_sc=False),
        out_type=[
            jax.ShapeDtypeStruct((E, DN), BF),
            jax.ShapeDtypeStruct((E, DN), BF),
        ],
        scratch_types=[
            pltpu.VMEM((nch, K), jnp.int32),
            pltpu.VMEM((nch, K), jnp.int32),
            pltpu.VMEM((K, DN), BF),
            pltpu.VMEM((K, DN), BF),
            pltpu.SemaphoreType.DMA,
            pltpu.SemaphoreType.DMA,
        ],
    )
    def sc_kernel(xb_hbm, src_hbm, dst_hbm, gs_hbm, gd_hbm,
                  src_v, dst_v, gs_v, gd_v, sem0, sem1):
        c = lax.axis_index("c")
        s = lax.axis_index("s")
        wid = s * NC + c
        pltpu.sync_copy(src_hbm.at[wid], src_v)
        pltpu.sync_copy(dst_hbm.at[wid], dst_v)
        ebase = wid * epw

        def body(j, carry):
            eo = ebase + j * K
            ga = pltpu.async_copy(xb_hbm.at[src_v.at[j]], gs_v, sem0)
            gb = pltpu.async_copy(xb_hbm.at[dst_v.at[j]], gd_v, sem1)
            ga.wait()
            pltpu.sync_copy(gs_v, gs_hbm.at[pl.ds(eo, K)])
            gb.wait()
            pltpu.sync_copy(gd_v, gd_hbm.at[pl.ds(eo, K)])
            return carry

        lax.fori_loop(0, nch, body, 0)

    return sc_kernel


def _sc_scatter(N, DN, E):
    assert E % (NW * K) == 0
    nch = E // (NW * K)
    epw = nch * K
    N2 = -(-N // (NS * 8)) * (NS * 8)
    rpt = N2 // NS

    @functools.partial(
        pl.kernel, mesh=_mesh(),
        compiler_params=pltpu.CompilerParams(use_tc_tiling_on_sc=False),
        out_type=[jax.ShapeDtypeStruct((NC, N2, DN), F32)],
        scratch_types=[
            pltpu.VMEM_SHARED((N2, DN), F32),
            pltpu.VMEM((nch, K), jnp.int32),
            pltpu.VMEM((K, DN), F32),
            pltpu.SemaphoreType.DMA,
        ],
    )
    def sc_kernel(msg_hbm, dst_hbm, zx_hbm, acc_hbm,
                  shx, dst_v, msg_v, sem0):
        c = lax.axis_index("c")
        s = lax.axis_index("s")
        wid = s * NC + c
        r0 = s * rpt
        pltpu.sync_copy(zx_hbm.at[pl.ds(r0, rpt)], shx.at[pl.ds(r0, rpt)])
        pltpu.sync_copy(dst_hbm.at[wid], dst_v)
        plsc.subcore_barrier()
        ebase = wid * epw

        def body(j, carry):
            pltpu.sync_copy(msg_hbm.at[pl.ds(ebase + j * K, K)], msg_v)
            pltpu.sync_copy(msg_v, shx.at[dst_v.at[j]], add=True)
            return carry

        lax.fori_loop(0, nch, body, 0)
        plsc.subcore_barrier()
        pltpu.sync_copy(shx.at[pl.ds(r0, rpt)], acc_hbm.at[c, pl.ds(r0, rpt)])

    return sc_kernel


def _full(shape):
    return pl.BlockSpec(shape, lambda i: (0,) * len(shape))


def _dot(a, b):
    return jnp.dot(a, b, preferred_element_type=F32)


def _tc_msgedge(E, DN, DE, BE=8000):
    nb = E // BE

    def body(gs_ref, gd_ref, e_ref, wn_ref, we_ref, wu_ref, bn_ref, be_ref,
             bu_ref, es_ref, msg_ref, ne_ref):
        gs = gs_ref[...]
        e = e_ref[...]
        msg_ref[...] = (_dot(gs, wn_ref[...]) + bn_ref[...]
                        + _dot(e, we_ref[...]) + be_ref[...])
        ec = jnp.concatenate([gs, gd_ref[...], e], axis=1)
        ne = _dot(ec, wu_ref[...]) + bu_ref[...] + es_ref[...]
        ne_ref[...] = jnp.maximum(ne, 0.0).astype(BF)

    return pl.pallas_call(
        body, grid=(nb,),
        in_specs=[pl.BlockSpec((BE, DN), lambda i: (i, 0)),
                  pl.BlockSpec((BE, DN), lambda i: (i, 0)),
                  pl.BlockSpec((BE, DE), lambda i: (i, 0)),
                  _full((DN, DN)), _full((DE, DN)),
                  _full((2 * DN + DE, DE)),
                  _full((1, DN)), _full((1, DN)), _full((1, DE)),
                  _full((1, 1))],
        out_specs=[pl.BlockSpec((BE, DN), lambda i: (i, 0)),
                   pl.BlockSpec((BE, DE), lambda i: (i, 0))],
        out_shape=[jax.ShapeDtypeStruct((E, DN), F32),
                   jax.ShapeDtypeStruct((E, DE), BF)],
    )


def _tc_node(N, DN, BN=1000):
    nb = N // BN

    def body(acc_ref, xb_ref, ws_ref, bs_ref, ns_ref, nx_ref):
        ax = acc_ref[0] + acc_ref[1]
        h = ax + (_dot(xb_ref[...], ws_ref[...]) + bs_ref[...]) + ns_ref[...]
        nx_ref[...] = jnp.maximum(h, 0.0).astype(BF)

    return pl.pallas_call(
        body, grid=(nb,),
        in_specs=[pl.BlockSpec((NC, BN, DN), lambda i: (0, i, 0)),
                  pl.BlockSpec((BN, DN), lambda i: (i, 0)),
                  _full((DN, DN)), _full((1, DN)), _full((1, 1))],
        out_specs=pl.BlockSpec((BN, DN), lambda i: (i, 0)),
        out_shape=jax.ShapeDtypeStruct((N, DN), BF),
    )


def _tc_head(E, DN, DE, HID, BE=8000):
    nb = E // BE

    def body(gs_ref, gd_ref, e_ref, w1_ref, b1_ref, w2_ref, b2_ref,
             w3_ref, b3_ref, out_ref):
        ec = jnp.concatenate([gs_ref[...], gd_ref[...], e_ref[...]], axis=1)
        h1 = jnp.maximum(_dot(ec, w1_ref[...]) + b1_ref[...], 0.0).astype(BF)
        h2 = jnp.maximum(_dot(h1, w2_ref[...]) + b2_ref[...], 0.0).astype(BF)
        z = _dot(h2, w3_ref[...]) + b3_ref[...]
        out_ref[...] = jax.nn.sigmoid(z)

    return pl.pallas_call(
        body, grid=(nb,),
        in_specs=[pl.BlockSpec((BE, DN), lambda i: (i, 0)),
                  pl.BlockSpec((BE, DN), lambda i: (i, 0)),
                  pl.BlockSpec((BE, DE), lambda i: (i, 0)),
                  _full((2 * DN + DE, HID)), _full((1, HID)),
                  _full((HID, HID)), _full((1, HID)),
                  _full((HID, 1)), _full((1, 1))],
        out_specs=pl.BlockSpec((BE, 1), lambda i: (i, 0)),
        out_shape=jax.ShapeDtypeStruct((E, 1), F32),
    )


def kernel(x, edge_index, edge_attr, params):
    N, DN = x.shape
    E = edge_index.shape[1]
    DE = edge_attr.shape[1]
    HID = params["mlp"]["W2"].shape[0]

    src = edge_index[0].astype(jnp.int32)
    dst = edge_index[1].astype(jnp.int32)
    nch = E // (NW * K)
    src3 = src.reshape(NW, nch, K)
    dst3 = dst.reshape(NW, nch, K)

    N2 = -(-N // (NS * 8)) * (NS * 8)
    zx = jnp.zeros((N2, DN), F32)

    sc_gather = _sc_gather2(N, DN, E)
    sc_scatter = _sc_scatter(N, DN, E)
    tc_msgedge = _tc_msgedge(E, DN, DE)
    tc_node = _tc_node(N, DN)

    xb = x.astype(BF)
    eb = edge_attr.astype(BF)
    for p in params["layers"]:
        gs, gd = sc_gather(xb, src3, dst3)
        msg, ne = tc_msgedge(
            gs, gd, eb, p["Wn"].astype(BF), p["We"].astype(BF),
            p["Wu"].astype(BF), p["bn"].reshape(1, DN), p["be"].reshape(1, DN),
            p["bu"].reshape(1, DE), p["edge_scalar"].reshape(1, 1))
        (acc,) = sc_scatter(msg, dst3, zx)
        xb = tc_node(acc, xb, p["Ws"].astype(BF), p["bs"].reshape(1, DN),
                     p["node_scalar"].reshape(1, 1))
        eb = ne

    m = params["mlp"]
    gs, gd = sc_gather(xb, src3, dst3)
    out = _tc_head(E, DN, DE, HID)(
        gs, gd, eb, m["W1"].astype(BF), m["b1"].reshape(1, HID),
        m["W2"].astype(BF), m["b2"].reshape(1, HID), m["W3"].astype(BF),
        m["b3"].reshape(1, 1))
    return out.reshape(E)


# speedup vs baseline: 1.4786x; 1.1073x over previous
import functools

import jax
import jax.numpy as jnp
from jax import lax
from jax.experimental import pallas as pl
from jax.experimental.pallas import tpu as pltpu
from jax.experimental.pallas import tpu_sc as plsc

NC, NS = 2, 16
NW = NC * NS
K = 80
BF = jnp.bfloat16
F32 = jnp.float32


def _mesh():
    return plsc.VectorSubcoreMesh(core_axis_name="c", subcore_axis_name="s",
                                  num_cores=NC, num_subcores=NS)


def _sc_gather2(N, DN, E):
    assert E % (NW * K) == 0
    nch = E // (NW * K)
    epw = nch * K

    @functools.partial(
        pl.kernel, mesh=_mesh(),
        compiler_params=pltpu.CompilerParams(use_tc_tiling_on_sc=False),
        out_type=[
            jax.ShapeDtypeStruct((E, DN), BF),
            jax.ShapeDtypeStruct((E, DN), BF),
        ],
        scratch_types=[
            pltpu.VMEM((nch, K), jnp.int32),
            pltpu.VMEM((nch, K), jnp.int32),
            pltpu.VMEM((2, K, DN), BF),
            pltpu.VMEM((2, K, DN), BF),
            pltpu.SemaphoreType.DMA,
            pltpu.SemaphoreType.DMA,
        ],
    )
    def sc_kernel(xb_hbm, src_hbm, dst_hbm, gs_hbm, gd_hbm,
                  src_v, dst_v, gs_v, gd_v, sem0, sem1):
        c = lax.axis_index("c")
        s = lax.axis_index("s")
        wid = s * NC + c
        pltpu.sync_copy(src_hbm.at[wid], src_v)
        pltpu.sync_copy(dst_hbm.at[wid], dst_v)
        ebase = wid * epw

        def issue(j, b):
            pltpu.async_copy(xb_hbm.at[src_v.at[j]], gs_v.at[b], sem0)
            pltpu.async_copy(xb_hbm.at[dst_v.at[j]], gd_v.at[b], sem1)

        def drain_store(j, b):
            eo = ebase + j * K
            pltpu.make_async_copy(xb_hbm.at[src_v.at[j]], gs_v.at[b],
                                  sem0).wait()
            pltpu.sync_copy(gs_v.at[b], gs_hbm.at[pl.ds(eo, K)])
            pltpu.make_async_copy(xb_hbm.at[dst_v.at[j]], gd_v.at[b],
                                  sem1).wait()
            pltpu.sync_copy(gd_v.at[b], gd_hbm.at[pl.ds(eo, K)])

        issue(0, 0)

        def body(j, carry):
            even = lax.rem(j, 2) == 0
            more = j + 1 < nch

            @pl.when(jnp.logical_and(even, more))
            def _():
                issue(j + 1, 1)

            @pl.when(jnp.logical_and(jnp.logical_not(even), more))
            def _():
                issue(j + 1, 0)

            @pl.when(even)
            def _():
                drain_store(j, 0)

            @pl.when(jnp.logical_not(even))
            def _():
                drain_store(j, 1)

            return carry

        lax.fori_loop(0, nch, body, 0)

    return sc_kernel


def _sc_scatter(N, DN, E):
    assert E % (NW * K) == 0
    nch = E // (NW * K)
    epw = nch * K
    N2 = -(-N // (NS * 8)) * (NS * 8)
    rpt = N2 // NS

    @functools.partial(
        pl.kernel, mesh=_mesh(),
        compiler_params=pltpu.CompilerParams(use_tc_tiling_on_sc=False),
        out_type=[jax.ShapeDtypeStruct((NC, N2, DN), F32)],
        scratch_types=[
            pltpu.VMEM_SHARED((N2, DN), F32),
            pltpu.VMEM((nch, K), jnp.int32),
            pltpu.VMEM((2, K, DN), F32),
            pltpu.SemaphoreType.DMA,
        ],
    )
    def sc_kernel(msg_hbm, dst_hbm, zx_hbm, acc_hbm,
                  shx, dst_v, msg_v, sem0):
        c = lax.axis_index("c")
        s = lax.axis_index("s")
        wid = s * NC + c
        r0 = s * rpt
        pltpu.sync_copy(zx_hbm.at[pl.ds(r0, rpt)], shx.at[pl.ds(r0, rpt)])
        pltpu.sync_copy(dst_hbm.at[wid], dst_v)
        plsc.subcore_barrier()
        ebase = wid * epw

        def issue(j, b):
            pltpu.async_copy(msg_hbm.at[pl.ds(ebase + j * K, K)],
                             msg_v.at[b], sem0)

        def drain_scatter(j, b):
            pltpu.make_async_copy(msg_hbm.at[pl.ds(ebase + j * K, K)],
                                  msg_v.at[b], sem0).wait()
            pltpu.sync_copy(msg_v.at[b], shx.at[dst_v.at[j]], add=True)

        issue(0, 0)

        def body(j, carry):
            even = lax.rem(j, 2) == 0
            more = j + 1 < nch

            @pl.when(jnp.logical_and(even, more))
            def _():
                issue(j + 1, 1)

            @pl.when(jnp.logical_and(jnp.logical_not(even), more))
            def _():
                issue(j + 1, 0)

            @pl.when(even)
            def _():
                drain_scatter(j, 0)

            @pl.when(jnp.logical_not(even))
            def _():
                drain_scatter(j, 1)

            return carry

        lax.fori_loop(0, nch, body, 0)
        plsc.subcore_barrier()
        pltpu.sync_copy(shx.at[pl.ds(r0, rpt)], acc_hbm.at[c, pl.ds(r0, rpt)])

    return sc_kernel


def _full(shape):
    return pl.BlockSpec(shape, lambda i: (0,) * len(shape))


def _dot(a, b):
    return jnp.dot(a, b, preferred_element_type=F32)


def _tc_msgedge(E, DN, DE, BE=8000):
    nb = E // BE

    def body(gs_ref, gd_ref, e_ref, wn_ref, we_ref, wu_ref, bn_ref, be_ref,
             bu_ref, es_ref, msg_ref, ne_ref):
        gs = gs_ref[...]
        e = e_ref[...]
        msg_ref[...] = (_dot(gs, wn_ref[...]) + bn_ref[...]
                        + _dot(e, we_ref[...]) + be_ref[...])
        ec = jnp.concatenate([gs, gd_ref[...], e], axis=1)
        ne = _dot(ec, wu_ref[...]) + bu_ref[...] + es_ref[...]
        ne_ref[...] = jnp.maximum(ne, 0.0).astype(BF)

    return pl.pallas_call(
        body, grid=(nb,),
        in_specs=[pl.BlockSpec((BE, DN), lambda i: (i, 0)),
                  pl.BlockSpec((BE, DN), lambda i: (i, 0)),
                  pl.BlockSpec((BE, DE), lambda i: (i, 0)),
                  _full((DN, DN)), _full((DE, DN)),
                  _full((2 * DN + DE, DE)),
                  _full((1, DN)), _full((1, DN)), _full((1, DE)),
                  _full((1, 1))],
        out_specs=[pl.BlockSpec((BE, DN), lambda i: (i, 0)),
                   pl.BlockSpec((BE, DE), lambda i: (i, 0))],
        out_shape=[jax.ShapeDtypeStruct((E, DN), F32),
                   jax.ShapeDtypeStruct((E, DE), BF)],
    )


def _tc_node(N, DN, BN=1000):
    nb = N // BN

    def body(acc_ref, xb_ref, ws_ref, bs_ref, ns_ref, nx_ref):
        ax = acc_ref[0] + acc_ref[1]
        h = ax + (_dot(xb_ref[...], ws_ref[...]) + bs_ref[...]) + ns_ref[...]
        nx_ref[...] = jnp.maximum(h, 0.0).astype(BF)

    return pl.pallas_call(
        body, grid=(nb,),
        in_specs=[pl.BlockSpec((NC, BN, DN), lambda i: (0, i, 0)),
                  pl.BlockSpec((BN, DN), lambda i: (i, 0)),
                  _full((DN, DN)), _full((1, DN)), _full((1, 1))],
        out_specs=pl.BlockSpec((BN, DN), lambda i: (i, 0)),
        out_shape=jax.ShapeDtypeStruct((N, DN), BF),
    )


def _tc_head(E, DN, DE, HID, BE=8000):
    nb = E // BE

    def body(gs_ref, gd_ref, e_ref, w1_ref, b1_ref, w2_ref, b2_ref,
             w3_ref, b3_ref, out_ref):
        ec = jnp.concatenate([gs_ref[...], gd_ref[...], e_ref[...]], axis=1)
        h1 = jnp.maximum(_dot(ec, w1_ref[...]) + b1_ref[...], 0.0).astype(BF)
        h2 = jnp.maximum(_dot(h1, w2_ref[...]) + b2_ref[...], 0.0).astype(BF)
        z = _dot(h2, w3_ref[...]) + b3_ref[...]
        out_ref[...] = jax.nn.sigmoid(z)

    return pl.pallas_call(
        body, grid=(nb,),
        in_specs=[pl.BlockSpec((BE, DN), lambda i: (i, 0)),
                  pl.BlockSpec((BE, DN), lambda i: (i, 0)),
                  pl.BlockSpec((BE, DE), lambda i: (i, 0)),
                  _full((2 * DN + DE, HID)), _full((1, HID)),
                  _full((HID, HID)), _full((1, HID)),
                  _full((HID, 1)), _full((1, 1))],
        out_specs=pl.BlockSpec((BE, 1), lambda i: (i, 0)),
        out_shape=jax.ShapeDtypeStruct((E, 1), F32),
    )


def kernel(x, edge_index, edge_attr, params):
    N, DN = x.shape
    E = edge_index.shape[1]
    DE = edge_attr.shape[1]
    HID = params["mlp"]["W2"].shape[0]

    src = edge_index[0].astype(jnp.int32)
    dst = edge_index[1].astype(jnp.int32)
    nch = E // (NW * K)
    src3 = src.reshape(NW, nch, K)
    dst3 = dst.reshape(NW, nch, K)

    N2 = -(-N // (NS * 8)) * (NS * 8)
    zx = jnp.zeros((N2, DN), F32)

    sc_gather = _sc_gather2(N, DN, E)
    sc_scatter = _sc_scatter(N, DN, E)
    tc_msgedge = _tc_msgedge(E, DN, DE)
    tc_node = _tc_node(N, DN)

    xb = x.astype(BF)
    eb = edge_attr.astype(BF)
    for p in params["layers"]:
        gs, gd = sc_gather(xb, src3, dst3)
        msg, ne = tc_msgedge(
            gs, gd, eb, p["Wn"].astype(BF), p["We"].astype(BF),
            p["Wu"].astype(BF), p["bn"].reshape(1, DN), p["be"].reshape(1, DN),
            p["bu"].reshape(1, DE), p["edge_scalar"].reshape(1, 1))
        (acc,) = sc_scatter(msg, dst3, zx)
        xb = tc_node(acc, xb, p["Ws"].astype(BF), p["bs"].reshape(1, DN),
                     p["node_scalar"].reshape(1, 1))
        eb = ne

    m = params["mlp"]
    gs, gd = sc_gather(xb, src3, dst3)
    out = _tc_head(E, DN, DE, HID)(
        gs, gd, eb, m["W1"].astype(BF), m["b1"].reshape(1, HID),
        m["W2"].astype(BF), m["b2"].reshape(1, HID), m["W3"].astype(BF),
        m["b3"].reshape(1, 1))
    return out.reshape(E)
